# baseline (speedup 1.0000x reference)
import dataclasses
import functools

import numpy as np

import jax
import jax.numpy as jnp
from jax import lax
from jax.experimental import pallas as pl
from jax.experimental.pallas import tpu as pltpu
from jax.experimental.pallas import tpu_sc as plsc

N = 10000
E = 320000
FEAT = 128
H = 128
C = 40

_F32 = jnp.float32
_BF16 = jnp.bfloat16
_sds = jax.ShapeDtypeStruct

_ILV = np.arange(H).reshape(H // 32, 2, 16).transpose(0, 2, 1).reshape(-1)


_ROWS = 2000
_NB = N // _ROWS


def _dotT(a, b):
    return lax.dot_general(a, b, (((1,), (1,)), ((), ())),
                           preferred_element_type=_F32,
                           precision=lax.Precision.HIGHEST)


def _tc0_body(x_ref, win_ref, bin_ref, w1_ref, w2_ref, wf_ref,
              h34_ref, h1_ref):
    q = _dotT(x_ref[...], win_ref[...]) + bin_ref[...]
    h34_ref[:, :H] = _dotT(q, w1_ref[...]).astype(_BF16)
    h34_ref[:, H:] = _dotT(q, w2_ref[...]).astype(_BF16)
    h1_ref[...] = jnp.maximum(_dotT(q, wf_ref[...]), 0.0)


def _norm_rows(xo):
    nrm = jnp.maximum(jnp.sqrt(jnp.sum(xo * xo, axis=1, keepdims=True)), 1e-12)
    return xo / nrm


def _tc1_body(h1_ref, p0_ref, p1_ref, w1_ref, w2_ref, wf_ref,
              h34_ref, h1o_ref):
    q = _norm_rows(h1_ref[...] + p0_ref[...] + p1_ref[...])
    h34_ref[:, :H] = _dotT(q, w1_ref[...]).astype(_BF16)
    h34_ref[:, H:] = _dotT(q, w2_ref[...]).astype(_BF16)
    h1o_ref[...] = jnp.maximum(_dotT(q, wf_ref[...]), 0.0)


def _tc2_body(h1_ref, p0_ref, p1_ref, wout_ref, bout_ref, o_ref):
    q = _norm_rows(h1_ref[...] + p0_ref[...] + p1_ref[...])
    logits = _dotT(q, wout_ref[...]) + bout_ref[...]
    m = jnp.max(logits, axis=1, keepdims=True)
    z = logits - m
    lse = jnp.log(jnp.sum(jnp.exp(z), axis=1, keepdims=True))
    o_ref[...] = z - lse


_row_spec = pl.BlockSpec((_ROWS, H), lambda i: (i, 0))
_row2_spec = pl.BlockSpec((_ROWS, 2 * H), lambda i: (i, 0))
_full_spec = pl.BlockSpec((H, H), lambda i: (0, 0))
_bias_spec = pl.BlockSpec((1, H), lambda i: (0, 0))

_mm_out_specs = [_row2_spec, _row_spec]
_mm_out_shape = [_sds((N, 2 * H), _BF16), _sds((N, H), _F32)]

_tc0 = pl.pallas_call(
    _tc0_body,
    grid=(_NB,),
    in_specs=[pl.BlockSpec((_ROWS, FEAT), lambda i: (i, 0)),
              pl.BlockSpec((H, FEAT), lambda i: (0, 0)),
              _bias_spec, _full_spec, _full_spec, _full_spec],
    out_specs=_mm_out_specs,
    out_shape=_mm_out_shape,
)

_tc1 = pl.pallas_call(
    _tc1_body,
    grid=(_NB,),
    in_specs=[_row_spec, _row_spec, _row_spec,
              _full_spec, _full_spec, _full_spec],
    out_specs=_mm_out_specs,
    out_shape=_mm_out_shape,
)

_tc2 = pl.pallas_call(
    _tc2_body,
    grid=(_NB,),
    in_specs=[_row_spec, _row_spec, _row_spec,
              pl.BlockSpec((C, H), lambda i: (0, 0)),
              pl.BlockSpec((1, C), lambda i: (0, 0))],
    out_specs=pl.BlockSpec((_ROWS, C), lambda i: (i, 0)),
    out_shape=_sds((N, C), _F32),
)


_NC = 2
_NS = 16
_NW = _NC * _NS
_EPW = E // _NW
_BC = 40
_NCH = _EPW // _BC
_DR = 624
_TAIL = N - _NS * _DR
_ZR = 16
_LC = H // 16
_NBK = H // 32

_mesh = plsc.VectorSubcoreMesh(core_axis_name="c", subcore_axis_name="s")

_sc_cp = pltpu.CompilerParams()
if "needs_layout_passes" in pltpu.CompilerParams.__dataclass_fields__:
    _sc_cp = dataclasses.replace(_sc_cp, needs_layout_passes=False)


def _unpk(v):
    w = plsc.bitcast(v, _BF16)
    return plsc.unpack(w, format=plsc.PackFormat.INTERLEAVED,
                       preferred_element_type=_F32)


@functools.partial(
    pl.kernel,
    mesh=_mesh,
    compiler_params=_sc_cp,
    out_type=_sds((_NC * N, H), _F32),
    scratch_types=[
        pltpu.VMEM((_BC,), jnp.int32),
        pltpu.VMEM((_BC,), jnp.int32),
        pltpu.VMEM((_BC,), jnp.int32),
        pltpu.VMEM((_BC,), jnp.int32),
        pltpu.VMEM((_BC, H), jnp.int32),
        pltpu.VMEM((_BC, H), jnp.int32),
        pltpu.VMEM((_BC, H), jnp.int32),
        pltpu.VMEM((_BC, H), jnp.int32),
        pltpu.VMEM((_BC, H), _F32),
        pltpu.VMEM((_BC, H), _F32),
        pltpu.VMEM((_BC,), jnp.int32),
        pltpu.VMEM((_BC,), jnp.int32),
        pltpu.VMEM((_ZR, H), _F32),
        pltpu.VMEM_SHARED((N, H), _F32),
        pltpu.SemaphoreType.DMA,
        pltpu.SemaphoreType.DMA,
        pltpu.SemaphoreType.DMA,
        pltpu.SemaphoreType.DMA,
        pltpu.SemaphoreType.DMA,
    ],
)
def _sc_edge(h34_hbm, row_hbm, col_hbm, out_hbm,
             rowi0, coli0, rowi1, coli1, g3r0, g3r1, g4c0, g4c1,
             msg0, msg1, scol0, scol1,
             zbuf, acc, sem_i, sem_g0, sem_g1, sem_s0, sem_s1):
    ci = lax.axis_index("c")
    si = lax.axis_index("s")
    ebase = (ci * _NS + si) * _EPW

    rowis = (rowi0, rowi1)
    colis = (coli0, coli1)
    g3rs = (g3r0, g3r1)
    g4cs = (g4c0, g4c1)
    msgs = (msg0, msg1)
    scols = (scol0, scol1)
    sgs = (sem_g0, sem_g1)
    sss = (sem_s0, sem_s1)

    @pl.loop(0, _ZR)
    def _(i):
        for j in range(_LC):
            zbuf[i, pl.ds(j * 16, 16)] = jnp.zeros((16,), _F32)

    @pl.loop(0, _DR // _ZR)
    def _(t):
        off = pl.multiple_of(si * _DR + t * _ZR, 8)
        pltpu.sync_copy(zbuf, acc.at[pl.ds(off, _ZR)])

    @pl.when(si == _NS - 1)
    def _():
        pltpu.sync_copy(zbuf.at[pl.ds(0, _TAIL)],
                        acc.at[pl.ds(_NS * _DR, _TAIL)])

    plsc.subcore_barrier()

    def issue_idx(k, s, sync):
        off = pl.multiple_of(ebase + k * _BC, 8)
        if sync:
            pltpu.sync_copy(row_hbm.at[pl.ds(off, _BC)], rowis[s])
            pltpu.sync_copy(col_hbm.at[pl.ds(off, _BC)], colis[s])
        else:
            pltpu.async_copy(row_hbm.at[pl.ds(off, _BC)], rowis[s], sem_i)
            pltpu.async_copy(col_hbm.at[pl.ds(off, _BC)], colis[s], sem_i)

    def wait_idx(s):
        off = pl.multiple_of(ebase, 8)
        pltpu.make_async_copy(row_hbm.at[pl.ds(off, _BC)], rowis[s], sem_i).wait()
        pltpu.make_async_copy(col_hbm.at[pl.ds(off, _BC)], colis[s], sem_i).wait()

    def issue_gathers(s):
        pltpu.async_copy(h34_hbm.at[rowis[s]], g3rs[s], sgs[s])
        pltpu.async_copy(h34_hbm.at[colis[s]], g4cs[s], sgs[s])

    def wait_gathers(s):
        pltpu.make_async_copy(h34_hbm.at[rowis[s]], g3rs[s], sgs[s]).wait()
        pltpu.make_async_copy(h34_hbm.at[colis[s]], g4cs[s], sgs[s]).wait()

    def wait_scatter(s):
        pltpu.make_async_copy(msgs[s], acc.at[scols[s]], sss[s]).wait()

    def compute_scatter(s):
        g3 = g3rs[s]
        g4 = g4cs[s]
        msg = msgs[s]
        scol = scols[s]

        @plsc.parallel_loop(0, _BC, unroll=8)
        def _(e):
            a3s = []
            b3s = []
            dots = None
            for j in range(_NBK):
                a3, b3 = _unpk(g3[e, pl.ds(16 * j, 16)])
                a4c, b4c = _unpk(g4[e, pl.ds(64 + 16 * j, 16)])
                term = a3 * a4c + b3 * b4c
                dots = term if dots is None else dots + term
                a3s.append(a3)
                b3s.append(b3)
            dvec = jnp.full((16,), jnp.sum(dots), _F32)
            svec = 1.0 / (1.0 + jnp.exp(dvec))
            for j in range(_NBK):
                a4, b4 = _unpk(g3[e, pl.ds(64 + 16 * j, 16)])
                msg[e, pl.ds(32 * j, 16)] = svec * (a3s[j] + a4) - a4
                msg[e, pl.ds(32 * j + 16, 16)] = svec * (b3s[j] + b4) - b4

        scol[pl.ds(0, 16)] = colis[s][pl.ds(0, 16)]
        scol[pl.ds(16, 16)] = colis[s][pl.ds(16, 16)]
        lanes = lax.broadcasted_iota(jnp.int32, (16,), 0)
        tidx = lanes + 32
        tmask = lanes < _BC - 32
        tv = plsc.load_gather(colis[s], [tidx], mask=tmask)
        plsc.store_scatter(scol, [tidx], tv, mask=tmask)
        pltpu.async_copy(msg, acc.at[scol], sss[s], add=True)

    issue_idx(0, 0, sync=True)
    issue_gathers(0)
    issue_idx(1, 1, sync=False)

    @pl.loop(0, _NCH, step=2)
    def _(c):
        def half(k, s):
            @pl.when(k + 1 < _NCH)
            def _():
                wait_idx(1 - s)
                issue_gathers(1 - s)

            wait_gathers(s)
            compute_scatter(s)

            @pl.when(k + 2 < _NCH)
            def _():
                issue_idx(k + 2, s, sync=False)

        half(c, 0)
        half(c + 1, 1)

    plsc.subcore_barrier()

    doff = pl.multiple_of(si * _DR, 8)
    ooff = pl.multiple_of(ci * N + si * _DR, 8)
    pltpu.sync_copy(acc.at[pl.ds(doff, _DR)], out_hbm.at[pl.ds(ooff, _DR)])

    @pl.when(si == _NS - 1)
    def _():
        toff = pl.multiple_of(ci * N + _NS * _DR, 8)
        pltpu.sync_copy(acc.at[pl.ds(_NS * _DR, _TAIL)],
                        out_hbm.at[pl.ds(toff, _TAIL)])


def kernel(x, edge_index, adj, ADJ1, W_in, b_in, W1, W2, Wf, W_out, b_out):
    row = edge_index[0].astype(jnp.int32)
    col = edge_index[1].astype(jnp.int32)
    b_in2 = b_in.reshape(1, H)
    b_out2 = b_out.reshape(1, C)
    W1p = W1[:, _ILV, :]
    W2p = W2[:, _ILV, :]

    def pack_words(h34):
        return lax.bitcast_convert_type(h34.reshape(N, 2 * H // 2, 2),
                                        jnp.int32)

    h34, h1 = _tc0(x, W_in, b_in2, W1p[0], W2p[0], Wf[0])
    p = _sc_edge(pack_words(h34), row, col)
    h34, h1 = _tc1(h1, p[:N], p[N:], W1p[1], W2p[1], Wf[1])
    p = _sc_edge(pack_words(h34), row, col)
    return _tc2(h1, p[:N], p[N:], W_out, b_out2)


# speedup vs baseline: 1.4981x; 1.4981x over previous
import dataclasses
import functools

import numpy as np

import jax
import jax.numpy as jnp
from jax import lax
from jax.experimental import pallas as pl
from jax.experimental.pallas import tpu as pltpu
from jax.experimental.pallas import tpu_sc as plsc

N = 10000
E = 320000
FEAT = 128
H = 128
C = 40

_F32 = jnp.float32
_BF16 = jnp.bfloat16
_sds = jax.ShapeDtypeStruct

_ILV = np.arange(H).reshape(H // 32, 2, 16).transpose(0, 2, 1).reshape(-1)


_ROWS = 2000
_NB = N // _ROWS


def _dotT(a, b):
    return lax.dot_general(a, b, (((1,), (1,)), ((), ())),
                           preferred_element_type=_F32,
                           precision=lax.Precision.HIGHEST)


def _tc0_body(x_ref, win_ref, bin_ref, w1_ref, w2_ref, wf_ref,
              h34_ref, h1_ref):
    q = _dotT(x_ref[...], win_ref[...]) + bin_ref[...]
    h34_ref[:, :H] = _dotT(q, w1_ref[...]).astype(_BF16)
    h34_ref[:, H:] = _dotT(q, w2_ref[...]).astype(_BF16)
    h1_ref[...] = jnp.maximum(_dotT(q, wf_ref[...]), 0.0)


def _norm_rows(xo):
    nrm = jnp.maximum(jnp.sqrt(jnp.sum(xo * xo, axis=1, keepdims=True)), 1e-12)
    return xo / nrm


def _tc1_body(h1_ref, p0_ref, p1_ref, w1_ref, w2_ref, wf_ref,
              h34_ref, h1o_ref):
    q = _norm_rows(h1_ref[...] + p0_ref[...] + p1_ref[...])
    h34_ref[:, :H] = _dotT(q, w1_ref[...]).astype(_BF16)
    h34_ref[:, H:] = _dotT(q, w2_ref[...]).astype(_BF16)
    h1o_ref[...] = jnp.maximum(_dotT(q, wf_ref[...]), 0.0)


def _tc2_body(h1_ref, p0_ref, p1_ref, wout_ref, bout_ref, o_ref):
    q = _norm_rows(h1_ref[...] + p0_ref[...] + p1_ref[...])
    logits = _dotT(q, wout_ref[...]) + bout_ref[...]
    m = jnp.max(logits, axis=1, keepdims=True)
    z = logits - m
    lse = jnp.log(jnp.sum(jnp.exp(z), axis=1, keepdims=True))
    o_ref[...] = z - lse


_row_spec = pl.BlockSpec((_ROWS, H), lambda i: (i, 0))
_row2_spec = pl.BlockSpec((_ROWS, 2 * H), lambda i: (i, 0))
_full_spec = pl.BlockSpec((H, H), lambda i: (0, 0))
_bias_spec = pl.BlockSpec((1, H), lambda i: (0, 0))

_mm_out_specs = [_row2_spec, _row_spec]
_mm_out_shape = [_sds((N, 2 * H), _BF16), _sds((N, H), _F32)]

_tc0 = pl.pallas_call(
    _tc0_body,
    grid=(_NB,),
    in_specs=[pl.BlockSpec((_ROWS, FEAT), lambda i: (i, 0)),
              pl.BlockSpec((H, FEAT), lambda i: (0, 0)),
              _bias_spec, _full_spec, _full_spec, _full_spec],
    out_specs=_mm_out_specs,
    out_shape=_mm_out_shape,
)

_tc1 = pl.pallas_call(
    _tc1_body,
    grid=(_NB,),
    in_specs=[_row_spec, _row_spec, _row_spec,
              _full_spec, _full_spec, _full_spec],
    out_specs=_mm_out_specs,
    out_shape=_mm_out_shape,
)

_tc2 = pl.pallas_call(
    _tc2_body,
    grid=(_NB,),
    in_specs=[_row_spec, _row_spec, _row_spec,
              pl.BlockSpec((C, H), lambda i: (0, 0)),
              pl.BlockSpec((1, C), lambda i: (0, 0))],
    out_specs=pl.BlockSpec((_ROWS, C), lambda i: (i, 0)),
    out_shape=_sds((N, C), _F32),
)


_NC = 2
_NS = 16
_NW = _NC * _NS
_EPW = E // _NW
_BC = 40
_NCH = _EPW // _BC
_DR = 624
_TAIL = N - _NS * _DR
_ZR = 16
_LC = H // 16
_NBK = H // 32

_mesh = plsc.VectorSubcoreMesh(core_axis_name="c", subcore_axis_name="s")

_sc_cp = pltpu.CompilerParams()
if "needs_layout_passes" in pltpu.CompilerParams.__dataclass_fields__:
    _sc_cp = dataclasses.replace(_sc_cp, needs_layout_passes=False)


def _unpk(v):
    w = plsc.bitcast(v, _BF16)
    return plsc.unpack(w, format=plsc.PackFormat.INTERLEAVED,
                       preferred_element_type=_F32)


@functools.partial(
    pl.kernel,
    mesh=_mesh,
    compiler_params=_sc_cp,
    out_type=_sds((_NC * N, H), _F32),
    scratch_types=[
        pltpu.VMEM((_BC,), jnp.int32),
        pltpu.VMEM((_BC,), jnp.int32),
        pltpu.VMEM((_BC,), jnp.int32),
        pltpu.VMEM((_BC,), jnp.int32),
        pltpu.VMEM((_BC, H), jnp.int32),
        pltpu.VMEM((_BC, H), jnp.int32),
        pltpu.VMEM((_BC, H), jnp.int32),
        pltpu.VMEM((_BC, H), jnp.int32),
        pltpu.VMEM((_BC, H), _F32),
        pltpu.VMEM((_BC, H), _F32),
        pltpu.VMEM((_BC,), jnp.int32),
        pltpu.VMEM((_BC,), jnp.int32),
        pltpu.VMEM((_ZR, H), _F32),
        pltpu.VMEM_SHARED((N, H), _F32),
        pltpu.SemaphoreType.DMA,
        pltpu.SemaphoreType.DMA,
        pltpu.SemaphoreType.DMA,
        pltpu.SemaphoreType.DMA,
        pltpu.SemaphoreType.DMA,
    ],
)
def _sc_edge(h34_hbm, row_hbm, col_hbm, out_hbm,
             rowi0, coli0, rowi1, coli1, g3r0, g3r1, g4c0, g4c1,
             msg0, msg1, scol0, scol1,
             zbuf, acc, sem_i, sem_g0, sem_g1, sem_s0, sem_s1):
    ci = lax.axis_index("c")
    si = lax.axis_index("s")
    ebase = (ci * _NS + si) * _EPW

    rowis = (rowi0, rowi1)
    colis = (coli0, coli1)
    g3rs = (g3r0, g3r1)
    g4cs = (g4c0, g4c1)
    msgs = (msg0, msg1)
    scols = (scol0, scol1)
    sgs = (sem_g0, sem_g1)
    sss = (sem_s0, sem_s1)

    @pl.loop(0, _ZR)
    def _(i):
        for j in range(_LC):
            zbuf[i, pl.ds(j * 16, 16)] = jnp.zeros((16,), _F32)

    @pl.loop(0, _DR // _ZR)
    def _(t):
        off = pl.multiple_of(si * _DR + t * _ZR, 8)
        pltpu.sync_copy(zbuf, acc.at[pl.ds(off, _ZR)])

    @pl.when(si == _NS - 1)
    def _():
        pltpu.sync_copy(zbuf.at[pl.ds(0, _TAIL)],
                        acc.at[pl.ds(_NS * _DR, _TAIL)])

    plsc.subcore_barrier()

    def issue_idx(k, s, sync):
        off = pl.multiple_of(ebase + k * _BC, 8)
        if sync:
            pltpu.sync_copy(row_hbm.at[pl.ds(off, _BC)], rowis[s])
            pltpu.sync_copy(col_hbm.at[pl.ds(off, _BC)], colis[s])
        else:
            pltpu.async_copy(row_hbm.at[pl.ds(off, _BC)], rowis[s], sem_i)
            pltpu.async_copy(col_hbm.at[pl.ds(off, _BC)], colis[s], sem_i)

    def wait_idx(s):
        off = pl.multiple_of(ebase, 8)
        pltpu.make_async_copy(row_hbm.at[pl.ds(off, _BC)], rowis[s], sem_i).wait()
        pltpu.make_async_copy(col_hbm.at[pl.ds(off, _BC)], colis[s], sem_i).wait()

    def issue_gathers(s):
        pltpu.async_copy(h34_hbm.at[rowis[s]], g3rs[s], sgs[s])
        pltpu.async_copy(h34_hbm.at[colis[s]], g4cs[s], sgs[s])

    def wait_gathers(s):
        pltpu.make_async_copy(h34_hbm.at[rowis[s]], g3rs[s], sgs[s]).wait()
        pltpu.make_async_copy(h34_hbm.at[colis[s]], g4cs[s], sgs[s]).wait()

    def wait_scatter(s):
        pltpu.make_async_copy(msgs[s], acc.at[scols[s]], sss[s]).wait()

    def compute_scatter(s):
        g3 = g3rs[s]
        g4 = g4cs[s]
        msg = msgs[s]
        scol = scols[s]

        @plsc.parallel_loop(0, _BC, unroll=2)
        def _(e):
            a3s = []
            b3s = []
            dots = None
            for j in range(_NBK):
                a3, b3 = _unpk(g3[e, pl.ds(16 * j, 16)])
                a4c, b4c = _unpk(g4[e, pl.ds(64 + 16 * j, 16)])
                term = a3 * a4c + b3 * b4c
                dots = term if dots is None else dots + term
                a3s.append(a3)
                b3s.append(b3)
            dvec = jnp.full((16,), jnp.sum(dots), _F32)
            svec = 1.0 / (1.0 + jnp.exp(dvec))
            for j in range(_NBK):
                a4, b4 = _unpk(g3[e, pl.ds(64 + 16 * j, 16)])
                msg[e, pl.ds(32 * j, 16)] = svec * (a3s[j] + a4) - a4
                msg[e, pl.ds(32 * j + 16, 16)] = svec * (b3s[j] + b4) - b4

        scol[pl.ds(0, 16)] = colis[s][pl.ds(0, 16)]
        scol[pl.ds(16, 16)] = colis[s][pl.ds(16, 16)]
        lanes = lax.broadcasted_iota(jnp.int32, (16,), 0)
        tidx = lanes + 32
        tmask = lanes < _BC - 32
        tv = plsc.load_gather(colis[s], [tidx], mask=tmask)
        plsc.store_scatter(scol, [tidx], tv, mask=tmask)
        pltpu.async_copy(msg, acc.at[scol], sss[s], add=True)

    issue_idx(0, 0, sync=True)
    issue_gathers(0)
    issue_idx(1, 1, sync=False)

    @pl.loop(0, _NCH, step=2)
    def _(c):
        def half(k, s):
            @pl.when(k + 1 < _NCH)
            def _():
                wait_idx(1 - s)
                issue_gathers(1 - s)

            wait_gathers(s)
            compute_scatter(s)

            @pl.when(k + 2 < _NCH)
            def _():
                issue_idx(k + 2, s, sync=False)

        half(c, 0)
        half(c + 1, 1)

    plsc.subcore_barrier()

    doff = pl.multiple_of(si * _DR, 8)
    ooff = pl.multiple_of(ci * N + si * _DR, 8)
    pltpu.sync_copy(acc.at[pl.ds(doff, _DR)], out_hbm.at[pl.ds(ooff, _DR)])

    @pl.when(si == _NS - 1)
    def _():
        toff = pl.multiple_of(ci * N + _NS * _DR, 8)
        pltpu.sync_copy(acc.at[pl.ds(_NS * _DR, _TAIL)],
                        out_hbm.at[pl.ds(toff, _TAIL)])


def kernel(x, edge_index, adj, ADJ1, W_in, b_in, W1, W2, Wf, W_out, b_out):
    row = edge_index[0].astype(jnp.int32)
    col = edge_index[1].astype(jnp.int32)
    b_in2 = b_in.reshape(1, H)
    b_out2 = b_out.reshape(1, C)
    W1p = W1[:, _ILV, :]
    W2p = W2[:, _ILV, :]

    def pack_words(h34):
        return lax.bitcast_convert_type(h34.reshape(N, 2 * H // 2, 2),
                                        jnp.int32)

    h34, h1 = _tc0(x, W_in, b_in2, W1p[0], W2p[0], Wf[0])
    p = _sc_edge(pack_words(h34), row, col)
    h34, h1 = _tc1(h1, p[:N], p[N:], W1p[1], W2p[1], Wf[1])
    p = _sc_edge(pack_words(h34), row, col)
    return _tc2(h1, p[:N], p[N:], W_out, b_out2)


# speedup vs baseline: 1.5256x; 1.0184x over previous
import dataclasses
import functools

import numpy as np

import jax
import jax.numpy as jnp
from jax import lax
from jax.experimental import pallas as pl
from jax.experimental.pallas import tpu as pltpu
from jax.experimental.pallas import tpu_sc as plsc

N = 10000
E = 320000
FEAT = 128
H = 128
C = 40

_F32 = jnp.float32
_BF16 = jnp.bfloat16
_sds = jax.ShapeDtypeStruct

_ILV = np.arange(H).reshape(H // 32, 2, 16).transpose(0, 2, 1).reshape(-1)


_ROWS = 2000
_NB = N // _ROWS


def _dotT(a, b):
    return lax.dot_general(a, b, (((1,), (1,)), ((), ())),
                           preferred_element_type=_F32,
                           precision=lax.Precision.HIGHEST)


def _tc0_body(x_ref, win_ref, bin_ref, w1_ref, w2_ref, wf_ref,
              h34_ref, h1_ref):
    q = _dotT(x_ref[...], win_ref[...]) + bin_ref[...]
    h34_ref[:, :H] = _dotT(q, w1_ref[...]).astype(_BF16)
    h34_ref[:, H:] = _dotT(q, w2_ref[...]).astype(_BF16)
    h1_ref[...] = jnp.maximum(_dotT(q, wf_ref[...]), 0.0)


def _norm_rows(xo):
    nrm = jnp.maximum(jnp.sqrt(jnp.sum(xo * xo, axis=1, keepdims=True)), 1e-12)
    return xo / nrm


def _tc1_body(h1_ref, p0_ref, p1_ref, w1_ref, w2_ref, wf_ref,
              h34_ref, h1o_ref):
    q = _norm_rows(h1_ref[...] + p0_ref[...] + p1_ref[...])
    h34_ref[:, :H] = _dotT(q, w1_ref[...]).astype(_BF16)
    h34_ref[:, H:] = _dotT(q, w2_ref[...]).astype(_BF16)
    h1o_ref[...] = jnp.maximum(_dotT(q, wf_ref[...]), 0.0)


def _tc2_body(h1_ref, p0_ref, p1_ref, wout_ref, bout_ref, o_ref):
    q = _norm_rows(h1_ref[...] + p0_ref[...] + p1_ref[...])
    logits = _dotT(q, wout_ref[...]) + bout_ref[...]
    m = jnp.max(logits, axis=1, keepdims=True)
    z = logits - m
    lse = jnp.log(jnp.sum(jnp.exp(z), axis=1, keepdims=True))
    o_ref[...] = z - lse


_row_spec = pl.BlockSpec((_ROWS, H), lambda i: (i, 0))
_row2_spec = pl.BlockSpec((_ROWS, 2 * H), lambda i: (i, 0))
_full_spec = pl.BlockSpec((H, H), lambda i: (0, 0))
_bias_spec = pl.BlockSpec((1, H), lambda i: (0, 0))

_mm_out_specs = [_row2_spec, _row_spec]
_mm_out_shape = [_sds((N, 2 * H), _BF16), _sds((N, H), _F32)]

_tc0 = pl.pallas_call(
    _tc0_body,
    grid=(_NB,),
    in_specs=[pl.BlockSpec((_ROWS, FEAT), lambda i: (i, 0)),
              pl.BlockSpec((H, FEAT), lambda i: (0, 0)),
              _bias_spec, _full_spec, _full_spec, _full_spec],
    out_specs=_mm_out_specs,
    out_shape=_mm_out_shape,
)

_tc1 = pl.pallas_call(
    _tc1_body,
    grid=(_NB,),
    in_specs=[_row_spec, _row_spec, _row_spec,
              _full_spec, _full_spec, _full_spec],
    out_specs=_mm_out_specs,
    out_shape=_mm_out_shape,
)

_tc2 = pl.pallas_call(
    _tc2_body,
    grid=(_NB,),
    in_specs=[_row_spec, _row_spec, _row_spec,
              pl.BlockSpec((C, H), lambda i: (0, 0)),
              pl.BlockSpec((1, C), lambda i: (0, 0))],
    out_specs=pl.BlockSpec((_ROWS, C), lambda i: (i, 0)),
    out_shape=_sds((N, C), _F32),
)


_NC = 2
_NS = 16
_NW = _NC * _NS
_EPW = E // _NW
_BC = 40
_NCH = _EPW // _BC
_DR = 624
_TAIL = N - _NS * _DR
_ZR = 16
_LC = H // 16
_NBK = H // 32

_mesh = plsc.VectorSubcoreMesh(core_axis_name="c", subcore_axis_name="s")

_sc_cp = pltpu.CompilerParams()
if "needs_layout_passes" in pltpu.CompilerParams.__dataclass_fields__:
    _sc_cp = dataclasses.replace(_sc_cp, needs_layout_passes=False)


def _unpk(v):
    w = plsc.bitcast(v, _BF16)
    return plsc.unpack(w, format=plsc.PackFormat.INTERLEAVED,
                       preferred_element_type=_F32)


@functools.partial(
    pl.kernel,
    mesh=_mesh,
    compiler_params=_sc_cp,
    out_type=_sds((_NC * N, H), _F32),
    scratch_types=[
        pltpu.VMEM((_BC,), jnp.int32),
        pltpu.VMEM((_BC,), jnp.int32),
        pltpu.VMEM((_BC,), jnp.int32),
        pltpu.VMEM((_BC,), jnp.int32),
        pltpu.VMEM((_BC, H), jnp.int32),
        pltpu.VMEM((_BC, H), jnp.int32),
        pltpu.VMEM((_BC, H), jnp.int32),
        pltpu.VMEM((_BC, H), jnp.int32),
        pltpu.VMEM((_BC, H), _F32),
        pltpu.VMEM((_BC, H), _F32),
        pltpu.VMEM((_BC,), jnp.int32),
        pltpu.VMEM((_BC,), jnp.int32),
        pltpu.VMEM((_ZR, H), _F32),
        pltpu.VMEM_SHARED((N, H), _F32),
        pltpu.SemaphoreType.DMA,
        pltpu.SemaphoreType.DMA,
        pltpu.SemaphoreType.DMA,
        pltpu.SemaphoreType.DMA,
        pltpu.SemaphoreType.DMA,
    ],
)
def _sc_edge(h34_hbm, row_hbm, col_hbm, out_hbm,
             rowi0, coli0, rowi1, coli1, g3r0, g3r1, g4c0, g4c1,
             msg0, msg1, scol0, scol1,
             zbuf, acc, sem_i, sem_g0, sem_g1, sem_s0, sem_s1):
    ci = lax.axis_index("c")
    si = lax.axis_index("s")
    ebase = (ci * _NS + si) * _EPW

    rowis = (rowi0, rowi1)
    colis = (coli0, coli1)
    g3rs = (g3r0, g3r1)
    g4cs = (g4c0, g4c1)
    msgs = (msg0, msg1)
    scols = (scol0, scol1)
    sgs = (sem_g0, sem_g1)
    sss = (sem_s0, sem_s1)

    @pl.loop(0, _ZR)
    def _(i):
        for j in range(_LC):
            zbuf[i, pl.ds(j * 16, 16)] = jnp.zeros((16,), _F32)

    @pl.loop(0, _DR // _ZR)
    def _(t):
        off = pl.multiple_of(si * _DR + t * _ZR, 8)
        pltpu.sync_copy(zbuf, acc.at[pl.ds(off, _ZR)])

    @pl.when(si == _NS - 1)
    def _():
        pltpu.sync_copy(zbuf.at[pl.ds(0, _TAIL)],
                        acc.at[pl.ds(_NS * _DR, _TAIL)])

    plsc.subcore_barrier()

    def issue_idx(k, s, sync):
        off = pl.multiple_of(ebase + k * _BC, 8)
        if sync:
            pltpu.sync_copy(row_hbm.at[pl.ds(off, _BC)], rowis[s])
            pltpu.sync_copy(col_hbm.at[pl.ds(off, _BC)], colis[s])
        else:
            pltpu.async_copy(row_hbm.at[pl.ds(off, _BC)], rowis[s], sem_i)
            pltpu.async_copy(col_hbm.at[pl.ds(off, _BC)], colis[s], sem_i)

    def wait_idx(s):
        off = pl.multiple_of(ebase, 8)
        pltpu.make_async_copy(row_hbm.at[pl.ds(off, _BC)], rowis[s], sem_i).wait()
        pltpu.make_async_copy(col_hbm.at[pl.ds(off, _BC)], colis[s], sem_i).wait()

    def issue_gathers(s):
        pltpu.async_copy(h34_hbm.at[rowis[s]], g3rs[s], sgs[s])
        pltpu.async_copy(h34_hbm.at[colis[s]], g4cs[s], sgs[s])

    def wait_gathers(s):
        pltpu.make_async_copy(h34_hbm.at[rowis[s]], g3rs[s], sgs[s]).wait()
        pltpu.make_async_copy(h34_hbm.at[colis[s]], g4cs[s], sgs[s]).wait()

    def wait_scatter(s):
        pltpu.make_async_copy(msgs[s], acc.at[scols[s]], sss[s]).wait()

    def compute_scatter(s):
        g3 = g3rs[s]
        g4 = g4cs[s]
        msg = msgs[s]
        scol = scols[s]

        @plsc.parallel_loop(0, _BC, unroll=1)
        def _(e):
            a3s = []
            b3s = []
            dots = None
            for j in range(_NBK):
                a3, b3 = _unpk(g3[e, pl.ds(16 * j, 16)])
                a4c, b4c = _unpk(g4[e, pl.ds(64 + 16 * j, 16)])
                term = a3 * a4c + b3 * b4c
                dots = term if dots is None else dots + term
                a3s.append(a3)
                b3s.append(b3)
            dvec = jnp.full((16,), jnp.sum(dots), _F32)
            svec = 1.0 / (1.0 + jnp.exp(dvec))
            for j in range(_NBK):
                a4, b4 = _unpk(g3[e, pl.ds(64 + 16 * j, 16)])
                msg[e, pl.ds(32 * j, 16)] = svec * (a3s[j] + a4) - a4
                msg[e, pl.ds(32 * j + 16, 16)] = svec * (b3s[j] + b4) - b4

        scol[pl.ds(0, 16)] = colis[s][pl.ds(0, 16)]
        scol[pl.ds(16, 16)] = colis[s][pl.ds(16, 16)]
        lanes = lax.broadcasted_iota(jnp.int32, (16,), 0)
        tidx = lanes + 32
        tmask = lanes < _BC - 32
        tv = plsc.load_gather(colis[s], [tidx], mask=tmask)
        plsc.store_scatter(scol, [tidx], tv, mask=tmask)
        pltpu.async_copy(msg, acc.at[scol], sss[s], add=True)

    issue_idx(0, 0, sync=True)
    issue_gathers(0)
    issue_idx(1, 1, sync=False)

    @pl.loop(0, _NCH, step=2)
    def _(c):
        def half(k, s):
            @pl.when(k + 1 < _NCH)
            def _():
                wait_idx(1 - s)
                issue_gathers(1 - s)

            wait_gathers(s)
            compute_scatter(s)

            @pl.when(k + 2 < _NCH)
            def _():
                issue_idx(k + 2, s, sync=False)

        half(c, 0)
        half(c + 1, 1)

    plsc.subcore_barrier()

    doff = pl.multiple_of(si * _DR, 8)
    ooff = pl.multiple_of(ci * N + si * _DR, 8)
    pltpu.sync_copy(acc.at[pl.ds(doff, _DR)], out_hbm.at[pl.ds(ooff, _DR)])

    @pl.when(si == _NS - 1)
    def _():
        toff = pl.multiple_of(ci * N + _NS * _DR, 8)
        pltpu.sync_copy(acc.at[pl.ds(_NS * _DR, _TAIL)],
                        out_hbm.at[pl.ds(toff, _TAIL)])


def kernel(x, edge_index, adj, ADJ1, W_in, b_in, W1, W2, Wf, W_out, b_out):
    row = edge_index[0].astype(jnp.int32)
    col = edge_index[1].astype(jnp.int32)
    b_in2 = b_in.reshape(1, H)
    b_out2 = b_out.reshape(1, C)
    W1p = W1[:, _ILV, :]
    W2p = W2[:, _ILV, :]

    def pack_words(h34):
        return lax.bitcast_convert_type(h34.reshape(N, 2 * H // 2, 2),
                                        jnp.int32)

    h34, h1 = _tc0(x, W_in, b_in2, W1p[0], W2p[0], Wf[0])
    p = _sc_edge(pack_words(h34), row, col)
    h34, h1 = _tc1(h1, p[:N], p[N:], W1p[1], W2p[1], Wf[1])
    p = _sc_edge(pack_words(h34), row, col)
    return _tc2(h1, p[:N], p[N:], W_out, b_out2)
